# baseline (speedup 1.0000x reference)
import functools

import jax
import jax.numpy as jnp
from jax import lax
from jax.experimental import pallas as pl
from jax.experimental.pallas import tpu as pltpu
from jax.experimental.pallas import tpu_sc as plsc

_B = 16384
_EMB = 64
_N_LABELS = 64
_TC_BLK = 2048
_K = 32
_NCH = 16


@functools.cache
def _make_gather2():
    info = plsc.get_sparse_core_info()
    nw = info.num_cores * info.num_subcores
    b_per_w = _B // nw
    mesh = plsc.VectorSubcoreMesh(core_axis_name="c", subcore_axis_name="s")

    @functools.partial(
        pl.kernel,
        mesh=mesh,
        compiler_params=pltpu.CompilerParams(use_tc_tiling_on_sc=True),
        out_type=[
            jax.ShapeDtypeStruct((_B, _EMB), jnp.float32),
            jax.ShapeDtypeStruct((_B, _EMB), jnp.float32),
        ],
        scratch_types=[
            pltpu.VMEM((b_per_w,), jnp.int32),
            pltpu.VMEM((b_per_w, _EMB), jnp.float32),
            pltpu.SemaphoreType.DMA,
        ],
    )
    def gather2(cells_hbm, genes_hbm, cell_tab3, gene_tab3,
                cell_out, gene_out,
                idx_v, stage, sem):
        wid = lax.axis_index("s") * info.num_cores + lax.axis_index("c")
        base = wid * b_per_w

        def one_table(ind_hbm, tab3, out_hbm):
            pltpu.sync_copy(ind_hbm.at[pl.ds(base, b_per_w)], idx_v)

            def fetch(g, _):
                vg = idx_v[pl.ds(g * 16, 16)]
                for k in range(16):
                    pltpu.async_copy(
                        tab3.at[vg[k]],
                        stage.at[g * 16 + k], sem)
                return 0

            lax.fori_loop(0, b_per_w // 16, fetch, 0)
            pltpu.make_async_copy(
                out_hbm.at[pl.ds(base, b_per_w)], stage, sem).wait()
            pltpu.sync_copy(stage, out_hbm.at[pl.ds(base, b_per_w)])

        one_table(cells_hbm, cell_tab3, cell_out)
        one_table(genes_hbm, gene_tab3, gene_out)

    return gather2


def _tc_body(ce_ref, ge_ref, w_ref, qz_ref, pz_ref, rec_ref):
    ce = ce_ref[...]
    ge = ge_ref[...]
    w = w_ref[...]
    pz_logit = lax.dot_general(ce, w, (((1,), (1,)), ((), ())),
                               preferred_element_type=jnp.float32)
    qz_logit = lax.dot_general(ce * ge, w, (((1,), (1,)), ((), ())),
                               preferred_element_type=jnp.float32)

    col = lax.broadcasted_iota(jnp.int32, qz_logit.shape, 1)
    row_max = jnp.max(qz_logit, axis=-1, keepdims=True)
    amax = jnp.min(jnp.where(qz_logit == row_max, col, _N_LABELS),
                   axis=-1, keepdims=True)
    onehot = (col == amax).astype(jnp.float32)
    rec_ref[...] = lax.dot_general(onehot, w, (((1,), (0,)), ((), ())),
                                   preferred_element_type=jnp.float32)

    qe = jnp.exp(qz_logit - row_max)
    qz_ref[...] = qe / jnp.sum(qe, axis=-1, keepdims=True)
    pe = jnp.exp(pz_logit - jnp.max(pz_logit, axis=-1, keepdims=True))
    pz_ref[...] = pe / jnp.sum(pe, axis=-1, keepdims=True)


def _tc_tail(ce, ge, w_ct):
    n_blk = _B // _TC_BLK
    blk = pl.BlockSpec((_TC_BLK, _EMB), lambda i: (i, 0))
    wspec = pl.BlockSpec((_N_LABELS, _EMB), lambda i: (0, 0))
    out = jax.ShapeDtypeStruct((_B, _N_LABELS), jnp.float32)
    out_e = jax.ShapeDtypeStruct((_B, _EMB), jnp.float32)
    return pl.pallas_call(
        _tc_body,
        grid=(n_blk,),
        in_specs=[blk, blk, wspec],
        out_specs=[pl.BlockSpec((_TC_BLK, _N_LABELS), lambda i: (i, 0))] * 2
        + [pl.BlockSpec((_TC_BLK, _EMB), lambda i: (i, 0))],
        out_shape=[out, out, out_e],
    )(ce, ge, w_ct)


def kernel(cells, genes, w_cell_table, w_gene_table, W_ct):
    ce, ge = _make_gather2()(cells, genes, w_cell_table, w_gene_table)
    qz, pz, recon = _tc_tail(ce, ge, W_ct)
    return (qz, pz, ce, recon)


# speedup vs baseline: 1.5769x; 1.5769x over previous
import functools

import jax
import jax.numpy as jnp
from jax import lax
from jax.experimental import pallas as pl
from jax.experimental.pallas import tpu as pltpu
from jax.experimental.pallas import tpu_sc as plsc

_B = 16384
_EMB = 64
_N_LABELS = 64
_TC_BLK = 2048


@functools.cache
def _make_gather2():
    info = plsc.get_sparse_core_info()
    nw = info.num_cores * info.num_subcores
    b_per_w = _B // nw
    mesh = plsc.VectorSubcoreMesh(core_axis_name="c", subcore_axis_name="s")

    @functools.partial(
        pl.kernel,
        mesh=mesh,
        out_type=[
            jax.ShapeDtypeStruct((_B, _EMB), jnp.float32),
            jax.ShapeDtypeStruct((_B, _EMB), jnp.float32),
        ],
        scratch_types=[
            pltpu.VMEM((b_per_w,), jnp.int32),
            pltpu.VMEM((b_per_w, _EMB), jnp.float32),
            pltpu.SemaphoreType.DMA,
        ],
    )
    def gather2(cells_hbm, genes_hbm, cell_tab, gene_tab,
                cell_out, gene_out,
                idx_v, stage, sem):
        wid = lax.axis_index("s") * info.num_cores + lax.axis_index("c")
        base = wid * b_per_w

        def one_table(ind_hbm, tab, out_hbm):
            pltpu.sync_copy(ind_hbm.at[pl.ds(base, b_per_w)], idx_v)

            def fetch(g, _):
                vg = idx_v[pl.ds(g * 16, 16)]
                tg = lax.shift_right_logical(vg, 3)
                sg = jnp.bitwise_and(vg, 7)
                for k in range(16):
                    pltpu.async_copy(
                        tab.at[tg[k], sg[k]],
                        stage.at[g * 16 + k], sem)
                return 0

            lax.fori_loop(0, b_per_w // 16, fetch, 0)
            pltpu.make_async_copy(
                out_hbm.at[pl.ds(base, b_per_w)], stage, sem).wait()
            pltpu.sync_copy(stage, out_hbm.at[pl.ds(base, b_per_w)])

        one_table(cells_hbm, cell_tab, cell_out)
        one_table(genes_hbm, gene_tab, gene_out)

    return gather2


def _tc_body(ce_ref, ge_ref, w_ref, eye_ref, qz_ref, pz_ref, ce_t_ref,
             rec_ref):
    ce = ce_ref[...]
    ge = ge_ref[...]
    w = w_ref[...]
    eye = eye_ref[...]
    pz_logit = lax.dot_general(w, ce, (((1,), (1,)), ((), ())),
                               preferred_element_type=jnp.float32)
    qz_logit = lax.dot_general(w, ce * ge, (((1,), (1,)), ((), ())),
                               preferred_element_type=jnp.float32)

    lab = lax.broadcasted_iota(jnp.int32, qz_logit.shape, 0)
    col_max = jnp.max(qz_logit, axis=0, keepdims=True)
    amax = jnp.min(jnp.where(qz_logit == col_max, lab, _N_LABELS),
                   axis=0, keepdims=True)
    onehot = (lab == amax).astype(jnp.float32)
    rec_ref[...] = lax.dot_general(w, onehot, (((0,), (0,)), ((), ())),
                                   preferred_element_type=jnp.float32)

    qe = jnp.exp(qz_logit - col_max)
    qz_ref[...] = qe / jnp.sum(qe, axis=0, keepdims=True)
    pe = jnp.exp(pz_logit - jnp.max(pz_logit, axis=0, keepdims=True))
    pz_ref[...] = pe / jnp.sum(pe, axis=0, keepdims=True)
    ce_t_ref[...] = lax.dot_general(eye, ce, (((1,), (1,)), ((), ())),
                                    preferred_element_type=jnp.float32)


def _tc_tail(ce, ge, w_ct):
    n_blk = _B // _TC_BLK
    blk = pl.BlockSpec((_TC_BLK, _EMB), lambda i: (i, 0))
    wspec = pl.BlockSpec((_N_LABELS, _EMB), lambda i: (0, 0))
    out_l = jax.ShapeDtypeStruct((_N_LABELS, _B), jnp.float32)
    out_e = jax.ShapeDtypeStruct((_EMB, _B), jnp.float32)
    eye = jnp.eye(_EMB, dtype=jnp.float32)
    return pl.pallas_call(
        _tc_body,
        grid=(n_blk,),
        in_specs=[blk, blk, wspec,
                  pl.BlockSpec((_EMB, _EMB), lambda i: (0, 0))],
        out_specs=[pl.BlockSpec((_N_LABELS, _TC_BLK), lambda i: (0, i))] * 2
        + [pl.BlockSpec((_EMB, _TC_BLK), lambda i: (0, i))] * 2,
        out_shape=[out_l, out_l, out_e, out_e],
    )(ce, ge, w_ct, eye)


def kernel(cells, genes, w_cell_table, w_gene_table, W_ct):
    ct3 = w_cell_table.reshape(-1, 8, _EMB)
    gt3 = w_gene_table.reshape(-1, 8, _EMB)
    ce, ge = _make_gather2()(cells, genes, ct3, gt3)
    qz_t, pz_t, ce_t, rec_t = _tc_tail(ce, ge, W_ct)
    return (qz_t.T, pz_t.T, ce_t.T, rec_t.T)


# speedup vs baseline: 1.5793x; 1.0015x over previous
import functools

import jax
import jax.numpy as jnp
from jax import lax
from jax.experimental import pallas as pl
from jax.experimental.pallas import tpu as pltpu
from jax.experimental.pallas import tpu_sc as plsc

_B = 16384
_EMB = 64
_N_LABELS = 64
_TC_BLK = 2048


@functools.cache
def _make_gather2():
    info = plsc.get_sparse_core_info()
    nw = info.num_cores * info.num_subcores
    b_per_w = _B // nw
    mesh = plsc.VectorSubcoreMesh(core_axis_name="c", subcore_axis_name="s")

    @functools.partial(
        pl.kernel,
        mesh=mesh,
        out_type=[
            jax.ShapeDtypeStruct((_B, _EMB), jnp.float32),
            jax.ShapeDtypeStruct((_B, _EMB), jnp.float32),
        ],
        scratch_types=[
            pltpu.VMEM((b_per_w,), jnp.int32),
            pltpu.VMEM((b_per_w,), jnp.int32),
            pltpu.VMEM((b_per_w // 2, _EMB), jnp.float32),
            pltpu.VMEM((b_per_w // 2, _EMB), jnp.float32),
            pltpu.SemaphoreType.DMA,
        ],
    )
    def gather2(cells_hbm, genes_hbm, cell_tab, gene_tab,
                cell_out, gene_out,
                cidx_v, gidx_v, cstage, gstage, sem):
        wid = lax.axis_index("s") * info.num_cores + lax.axis_index("c")
        base = wid * b_per_w

        pltpu.sync_copy(cells_hbm.at[pl.ds(base, b_per_w)], cidx_v)
        pltpu.sync_copy(genes_hbm.at[pl.ds(base, b_per_w)], gidx_v)
        half = b_per_w // 2

        for p in range(2):
            def fetch(g, _, p=p):
                off = p * half
                cv = cidx_v[pl.ds(off + g * 16, 16)]
                gv = gidx_v[pl.ds(off + g * 16, 16)]
                ct = lax.shift_right_logical(cv, 3)
                cs = jnp.bitwise_and(cv, 7)
                gt = lax.shift_right_logical(gv, 3)
                gs = jnp.bitwise_and(gv, 7)
                for k in range(16):
                    pltpu.async_copy(
                        cell_tab.at[ct[k], cs[k]],
                        cstage.at[g * 16 + k], sem)
                    pltpu.async_copy(
                        gene_tab.at[gt[k], gs[k]],
                        gstage.at[g * 16 + k], sem)
                return 0

            lax.fori_loop(0, half // 16, fetch, 0)
            pltpu.make_async_copy(
                cell_out.at[pl.ds(base + p * half, half)], cstage, sem).wait()
            pltpu.make_async_copy(
                gene_out.at[pl.ds(base + p * half, half)], gstage, sem).wait()
            pltpu.sync_copy(cstage, cell_out.at[pl.ds(base + p * half, half)])
            pltpu.sync_copy(gstage, gene_out.at[pl.ds(base + p * half, half)])

    return gather2


def _tc_body(ce_ref, ge_ref, w_ref, eye_ref, qz_ref, pz_ref, ce_t_ref,
             rec_ref):
    ce = ce_ref[...]
    ge = ge_ref[...]
    w = w_ref[...]
    eye = eye_ref[...]
    pz_logit = lax.dot_general(w, ce, (((1,), (1,)), ((), ())),
                               preferred_element_type=jnp.float32)
    qz_logit = lax.dot_general(w, ce * ge, (((1,), (1,)), ((), ())),
                               preferred_element_type=jnp.float32)

    lab = lax.broadcasted_iota(jnp.int32, qz_logit.shape, 0)
    col_max = jnp.max(qz_logit, axis=0, keepdims=True)
    amax = jnp.min(jnp.where(qz_logit == col_max, lab, _N_LABELS),
                   axis=0, keepdims=True)
    onehot = (lab == amax).astype(jnp.float32)
    rec_ref[...] = lax.dot_general(w, onehot, (((0,), (0,)), ((), ())),
                                   preferred_element_type=jnp.float32)

    qe = jnp.exp(qz_logit - col_max)
    qz_ref[...] = qe / jnp.sum(qe, axis=0, keepdims=True)
    pe = jnp.exp(pz_logit - jnp.max(pz_logit, axis=0, keepdims=True))
    pz_ref[...] = pe / jnp.sum(pe, axis=0, keepdims=True)
    ce_t_ref[...] = lax.dot_general(eye, ce, (((1,), (1,)), ((), ())),
                                    preferred_element_type=jnp.float32)


def _tc_tail(ce, ge, w_ct):
    n_blk = _B // _TC_BLK
    blk = pl.BlockSpec((_TC_BLK, _EMB), lambda i: (i, 0))
    wspec = pl.BlockSpec((_N_LABELS, _EMB), lambda i: (0, 0))
    out_l = jax.ShapeDtypeStruct((_N_LABELS, _B), jnp.float32)
    out_e = jax.ShapeDtypeStruct((_EMB, _B), jnp.float32)
    eye = jnp.eye(_EMB, dtype=jnp.float32)
    return pl.pallas_call(
        _tc_body,
        grid=(n_blk,),
        in_specs=[blk, blk, wspec,
                  pl.BlockSpec((_EMB, _EMB), lambda i: (0, 0))],
        out_specs=[pl.BlockSpec((_N_LABELS, _TC_BLK), lambda i: (0, i))] * 2
        + [pl.BlockSpec((_EMB, _TC_BLK), lambda i: (0, i))] * 2,
        out_shape=[out_l, out_l, out_e, out_e],
    )(ce, ge, w_ct, eye)


def kernel(cells, genes, w_cell_table, w_gene_table, W_ct):
    ct3 = w_cell_table.reshape(-1, 8, _EMB)
    gt3 = w_gene_table.reshape(-1, 8, _EMB)
    ce, ge = _make_gather2()(cells, genes, ct3, gt3)
    qz_t, pz_t, ce_t, rec_t = _tc_tail(ce, ge, W_ct)
    return (qz_t.T, pz_t.T, ce_t.T, rec_t.T)


# speedup vs baseline: 1.5879x; 1.0054x over previous
import functools

import jax
import jax.numpy as jnp
from jax import lax
from jax.experimental import pallas as pl
from jax.experimental.pallas import tpu as pltpu
from jax.experimental.pallas import tpu_sc as plsc

_B = 16384
_EMB = 64
_N_LABELS = 64
_TC_BLK = 4096


@functools.cache
def _make_gather2():
    info = plsc.get_sparse_core_info()
    nw = info.num_cores * info.num_subcores
    b_per_w = _B // nw
    mesh = plsc.VectorSubcoreMesh(core_axis_name="c", subcore_axis_name="s")

    @functools.partial(
        pl.kernel,
        mesh=mesh,
        out_type=[
            jax.ShapeDtypeStruct((_B, _EMB), jnp.float32),
            jax.ShapeDtypeStruct((_B, _EMB), jnp.float32),
        ],
        scratch_types=[
            pltpu.VMEM((b_per_w,), jnp.int32),
            pltpu.VMEM((b_per_w,), jnp.int32),
            pltpu.VMEM((b_per_w // 2, _EMB), jnp.float32),
            pltpu.VMEM((b_per_w // 2, _EMB), jnp.float32),
            pltpu.SemaphoreType.DMA,
        ],
    )
    def gather2(cells_hbm, genes_hbm, cell_tab, gene_tab,
                cell_out, gene_out,
                cidx_v, gidx_v, cstage, gstage, sem):
        wid = lax.axis_index("s") * info.num_cores + lax.axis_index("c")
        base = wid * b_per_w

        pltpu.sync_copy(cells_hbm.at[pl.ds(base, b_per_w)], cidx_v)
        pltpu.sync_copy(genes_hbm.at[pl.ds(base, b_per_w)], gidx_v)
        half = b_per_w // 2

        for p in range(2):
            def fetch(g, _, p=p):
                off = p * half
                cv = cidx_v[pl.ds(off + g * 16, 16)]
                gv = gidx_v[pl.ds(off + g * 16, 16)]
                ct = lax.shift_right_logical(cv, 3)
                cs = jnp.bitwise_and(cv, 7)
                gt = lax.shift_right_logical(gv, 3)
                gs = jnp.bitwise_and(gv, 7)
                for k in range(16):
                    pltpu.async_copy(
                        cell_tab.at[ct[k], cs[k]],
                        cstage.at[g * 16 + k], sem)
                    pltpu.async_copy(
                        gene_tab.at[gt[k], gs[k]],
                        gstage.at[g * 16 + k], sem)
                return 0

            lax.fori_loop(0, half // 16, fetch, 0)
            pltpu.make_async_copy(
                cell_out.at[pl.ds(base + p * half, half)], cstage, sem).wait()
            pltpu.make_async_copy(
                gene_out.at[pl.ds(base + p * half, half)], gstage, sem).wait()
            pltpu.sync_copy(cstage, cell_out.at[pl.ds(base + p * half, half)])
            pltpu.sync_copy(gstage, gene_out.at[pl.ds(base + p * half, half)])

    return gather2


def _tc_body(ce_ref, ge_ref, w_ref, eye_ref, qz_ref, pz_ref, ce_t_ref,
             rec_ref):
    ce = ce_ref[...]
    ge = ge_ref[...]
    w = w_ref[...]
    eye = eye_ref[...]
    pz_logit = lax.dot_general(w, ce, (((1,), (1,)), ((), ())),
                               preferred_element_type=jnp.float32)
    qz_logit = lax.dot_general(w, ce * ge, (((1,), (1,)), ((), ())),
                               preferred_element_type=jnp.float32)

    lab = lax.broadcasted_iota(jnp.int32, qz_logit.shape, 0)
    col_max = jnp.max(qz_logit, axis=0, keepdims=True)
    amax = jnp.min(jnp.where(qz_logit == col_max, lab, _N_LABELS),
                   axis=0, keepdims=True)
    onehot = (lab == amax).astype(jnp.float32)
    rec_ref[...] = lax.dot_general(w, onehot, (((0,), (0,)), ((), ())),
                                   preferred_element_type=jnp.float32)

    qe = jnp.exp(qz_logit - col_max)
    qz_ref[...] = qe / jnp.sum(qe, axis=0, keepdims=True)
    pe = jnp.exp(pz_logit - jnp.max(pz_logit, axis=0, keepdims=True))
    pz_ref[...] = pe / jnp.sum(pe, axis=0, keepdims=True)
    ce_t_ref[...] = lax.dot_general(eye, ce, (((1,), (1,)), ((), ())),
                                    preferred_element_type=jnp.float32)


def _tc_tail(ce, ge, w_ct):
    n_blk = _B // _TC_BLK
    blk = pl.BlockSpec((_TC_BLK, _EMB), lambda i: (i, 0))
    wspec = pl.BlockSpec((_N_LABELS, _EMB), lambda i: (0, 0))
    out_l = jax.ShapeDtypeStruct((_N_LABELS, _B), jnp.float32)
    out_e = jax.ShapeDtypeStruct((_EMB, _B), jnp.float32)
    eye = jnp.eye(_EMB, dtype=jnp.float32)
    return pl.pallas_call(
        _tc_body,
        grid=(n_blk,),
        in_specs=[blk, blk, wspec,
                  pl.BlockSpec((_EMB, _EMB), lambda i: (0, 0))],
        out_specs=[pl.BlockSpec((_N_LABELS, _TC_BLK), lambda i: (0, i))] * 2
        + [pl.BlockSpec((_EMB, _TC_BLK), lambda i: (0, i))] * 2,
        out_shape=[out_l, out_l, out_e, out_e],
    )(ce, ge, w_ct, eye)


def kernel(cells, genes, w_cell_table, w_gene_table, W_ct):
    ct3 = w_cell_table.reshape(-1, 8, _EMB)
    gt3 = w_gene_table.reshape(-1, 8, _EMB)
    ce, ge = _make_gather2()(cells, genes, ct3, gt3)
    qz_t, pz_t, ce_t, rec_t = _tc_tail(ce, ge, W_ct)
    return (qz_t.T, pz_t.T, ce_t.T, rec_t.T)
